# baseline (speedup 1.0000x reference)
import functools

import jax
import jax.numpy as jnp
from jax import lax
from jax.experimental import pallas as pl
from jax.experimental.pallas import tpu as pltpu
from jax.experimental.pallas import tpu_sc as plsc

NODE_DIM = 128
FT_DIM = 128
EDGE_DIM = 16
N_NODES = 100000
N_EDGES = 1600000
N_ATOM = 100

BN = 2000
NB = N_NODES // BN

CHUNK = 128
N_CHUNKS = N_EDGES // CHUNK
NW = 32
W_CHUNKS = N_CHUNKS // NW
EXTRA = N_CHUNKS - W_CHUNKS * NW
K = 13
GROUPS = W_CHUNKS // K
GROUP_ROWS = K * CHUNK


def _node_body(x_ref, atoms_ref, at_ref, w_ref, b_ref, o_ref):
    a = atoms_ref[0, 0, :]
    onehot = (a[:, None] == lax.broadcasted_iota(jnp.int32, (BN, N_ATOM), 1)
              ).astype(jnp.float32)
    w = w_ref[...]
    p = lax.dot_general(at_ref[...], w[:, :NODE_DIM],
                        (((1,), (1,)), ((), ())),
                        preferred_element_type=jnp.float32)
    t1 = lax.dot_general(onehot, p, (((1,), (0,)), ((), ())),
                         preferred_element_type=jnp.float32)
    t2 = lax.dot_general(x_ref[...], w[:, NODE_DIM:],
                         (((1,), (1,)), ((), ())),
                         preferred_element_type=jnp.float32)
    o_ref[...] = t1 + t2 + b_ref[...]


def _node_proj(x_features, atoms, atom_table, W, b):
    atoms3 = atoms.reshape(NB, 1, BN)
    b2 = b.reshape(1, NODE_DIM)
    return pl.pallas_call(
        _node_body,
        grid=(NB,),
        in_specs=[
            pl.BlockSpec((BN, FT_DIM), lambda i: (i, 0)),
            pl.BlockSpec((1, 1, BN), lambda i: (i, 0, 0)),
            pl.BlockSpec((N_ATOM, NODE_DIM), lambda i: (0, 0)),
            pl.BlockSpec((NODE_DIM, NODE_DIM + FT_DIM), lambda i: (0, 0)),
            pl.BlockSpec((1, NODE_DIM), lambda i: (0, 0)),
        ],
        out_specs=pl.BlockSpec((BN, NODE_DIM), lambda i: (i, 0)),
        out_shape=jax.ShapeDtypeStruct((N_NODES, NODE_DIM), jnp.float32),
    )(x_features, atoms3, atom_table, W, b2)


REP_STRIDE = 50 * EDGE_DIM + 1


def _edge_gather(edge_table, edge_attr):
    rep = jnp.broadcast_to(edge_table.reshape(1, 50 * EDGE_DIM),
                           (16, 50 * EDGE_DIM))
    tbl_flat = jnp.pad(rep, ((0, 0), (0, 1))).reshape(16 * REP_STRIDE)
    mesh = plsc.VectorSubcoreMesh(core_axis_name="c", subcore_axis_name="s")

    @functools.partial(
        pl.kernel,
        mesh=mesh,
        out_type=jax.ShapeDtypeStruct((EDGE_DIM, N_EDGES), jnp.float32),
        scratch_types=[
            pltpu.VMEM((16 * REP_STRIDE,), jnp.float32),
            pltpu.VMEM(((W_CHUNKS + 1) * CHUNK,), jnp.int32),
            pltpu.VMEM((EDGE_DIM, GROUP_ROWS), jnp.float32),
            pltpu.VMEM((EDGE_DIM, GROUP_ROWS), jnp.float32),
            pltpu.SemaphoreType.DMA,
            pltpu.SemaphoreType.DMA,
        ],
        compiler_params=pltpu.CompilerParams(needs_layout_passes=False),
    )
    def k(table_hbm, idx_hbm, out_hbm, tbl_v, idx_v, cols_a, cols_b,
          sem_a, sem_b):
        wid = lax.axis_index("s") * 2 + jax.lax.axis_index("c")
        cbase = wid * W_CHUNKS + jnp.minimum(wid, EXTRA)
        ebase0 = cbase * CHUNK
        lane_base = lax.iota(jnp.int32, 16) * REP_STRIDE
        cols = [jnp.full((16,), j, jnp.int32) for j in range(EDGE_DIM)]

        pltpu.sync_copy(table_hbm, tbl_v)
        pltpu.sync_copy(idx_hbm.at[pl.ds(ebase0, W_CHUNKS * CHUNK)],
                        idx_v.at[pl.ds(0, W_CHUNKS * CHUNK)])

        def fill(g, cols_v):
            @plsc.parallel_loop(0, GROUP_ROWS // 16)
            def _blk(i16):
                e16 = (idx_v[pl.ds(g * GROUP_ROWS + i16 * 16, 16)] * EDGE_DIM
                       + lane_base)
                for j in range(EDGE_DIM):
                    cols_v[j, pl.ds(i16 * 16, 16)] = plsc.load_gather(
                        tbl_v, [e16 + cols[j]])

        def out_slice(g, n=GROUP_ROWS):
            return out_hbm.at[:, pl.ds(ebase0 + g * GROUP_ROWS, n)]

        def body(t, carry):
            @pl.when(t > 0)
            def _():
                pltpu.make_async_copy(cols_a, out_slice(0), sem_a).wait()
            fill(2 * t, cols_a)
            pltpu.async_copy(cols_a, out_slice(2 * t), sem_a)

            @pl.when(t > 0)
            def _():
                pltpu.make_async_copy(cols_b, out_slice(0), sem_b).wait()
            fill(2 * t + 1, cols_b)
            pltpu.async_copy(cols_b, out_slice(2 * t + 1), sem_b)
            return carry

        lax.fori_loop(0, GROUPS // 2, body, 0)
        pltpu.make_async_copy(cols_a, out_slice(0), sem_a).wait()
        pltpu.make_async_copy(cols_b, out_slice(0), sem_b).wait()

        @pl.when(wid < EXTRA)
        def _tail():
            pltpu.sync_copy(idx_hbm.at[pl.ds(ebase0 + W_CHUNKS * CHUNK,
                                             CHUNK)],
                            idx_v.at[pl.ds(W_CHUNKS * CHUNK, CHUNK)])

            @plsc.parallel_loop(0, CHUNK // 16)
            def _blk(i16):
                e16 = (idx_v[pl.ds(W_CHUNKS * CHUNK + i16 * 16, 16)]
                       * EDGE_DIM + lane_base)
                for j in range(EDGE_DIM):
                    cols_a[j, pl.ds(i16 * 16, 16)] = plsc.load_gather(
                        tbl_v, [e16 + cols[j]])

            pltpu.sync_copy(cols_a.at[:, pl.ds(0, CHUNK)],
                            out_hbm.at[:, pl.ds(ebase0 + W_CHUNKS * CHUNK,
                                                CHUNK)])

    return k(tbl_flat, edge_attr).T


def kernel(x_features, atoms, edge_attr, pos, atom_table, edge_table, W, b):
    del pos
    x_out = _node_proj(x_features, atoms.astype(jnp.int32), atom_table, W, b)
    edge_embed = _edge_gather(edge_table, edge_attr.astype(jnp.int32))
    return (x_out, edge_embed)


# speedup vs baseline: 1.0870x; 1.0870x over previous
import functools

import jax
import jax.numpy as jnp
from jax import lax
from jax.experimental import pallas as pl
from jax.experimental.pallas import tpu as pltpu
from jax.experimental.pallas import tpu_sc as plsc

NODE_DIM = 128
FT_DIM = 128
EDGE_DIM = 16
N_NODES = 100000
N_EDGES = 1600000
N_ATOM = 100

BN = 4000
NB = N_NODES // BN

CHUNK = 128
N_CHUNKS = N_EDGES // CHUNK
NW = 32
W_CHUNKS = N_CHUNKS // NW
EXTRA = N_CHUNKS - W_CHUNKS * NW
K = 13
GROUPS = W_CHUNKS // K
GROUP_ROWS = K * CHUNK


def _node_body(x_ref, atoms_ref, at_ref, w_ref, b_ref, o_ref):
    a = atoms_ref[0, 0, :]
    onehot = (a[:, None] == lax.broadcasted_iota(jnp.int32, (BN, N_ATOM), 1)
              ).astype(jnp.float32)
    w = w_ref[...]
    p = lax.dot_general(at_ref[...], w[:, :NODE_DIM],
                        (((1,), (1,)), ((), ())),
                        preferred_element_type=jnp.float32)
    t1 = lax.dot_general(onehot, p, (((1,), (0,)), ((), ())),
                         preferred_element_type=jnp.float32)
    t2 = lax.dot_general(x_ref[...], w[:, NODE_DIM:],
                         (((1,), (1,)), ((), ())),
                         preferred_element_type=jnp.float32)
    o_ref[...] = t1 + t2 + b_ref[...]


def _node_proj(x_features, atoms, atom_table, W, b):
    atoms3 = atoms.reshape(NB, 1, BN)
    b2 = b.reshape(1, NODE_DIM)
    return pl.pallas_call(
        _node_body,
        grid=(NB,),
        in_specs=[
            pl.BlockSpec((BN, FT_DIM), lambda i: (i, 0)),
            pl.BlockSpec((1, 1, BN), lambda i: (i, 0, 0)),
            pl.BlockSpec((N_ATOM, NODE_DIM), lambda i: (0, 0)),
            pl.BlockSpec((NODE_DIM, NODE_DIM + FT_DIM), lambda i: (0, 0)),
            pl.BlockSpec((1, NODE_DIM), lambda i: (0, 0)),
        ],
        out_specs=pl.BlockSpec((BN, NODE_DIM), lambda i: (i, 0)),
        out_shape=jax.ShapeDtypeStruct((N_NODES, NODE_DIM), jnp.float32),
    )(x_features, atoms3, atom_table, W, b2)


REP_STRIDE = 50 * EDGE_DIM + 1


def _edge_gather(edge_table, edge_attr):
    rep = jnp.broadcast_to(edge_table.reshape(1, 50 * EDGE_DIM),
                           (16, 50 * EDGE_DIM))
    tbl_flat = jnp.pad(rep, ((0, 0), (0, 1))).reshape(16 * REP_STRIDE)
    mesh = plsc.VectorSubcoreMesh(core_axis_name="c", subcore_axis_name="s")

    @functools.partial(
        pl.kernel,
        mesh=mesh,
        out_type=jax.ShapeDtypeStruct((EDGE_DIM, N_EDGES), jnp.float32),
        scratch_types=[
            pltpu.VMEM((16 * REP_STRIDE,), jnp.float32),
            pltpu.VMEM(((W_CHUNKS + 1) * CHUNK,), jnp.int32),
            pltpu.VMEM((EDGE_DIM, GROUP_ROWS), jnp.float32),
            pltpu.VMEM((EDGE_DIM, GROUP_ROWS), jnp.float32),
            pltpu.SemaphoreType.DMA,
            pltpu.SemaphoreType.DMA,
        ],
        compiler_params=pltpu.CompilerParams(needs_layout_passes=False),
    )
    def k(table_hbm, idx_hbm, out_hbm, tbl_v, idx_v, cols_a, cols_b,
          sem_a, sem_b):
        wid = lax.axis_index("s") * 2 + jax.lax.axis_index("c")
        cbase = wid * W_CHUNKS + jnp.minimum(wid, EXTRA)
        ebase0 = cbase * CHUNK
        lane_base = lax.iota(jnp.int32, 16) * REP_STRIDE
        cols = [jnp.full((16,), j, jnp.int32) for j in range(EDGE_DIM)]

        pltpu.sync_copy(table_hbm, tbl_v)
        pltpu.sync_copy(idx_hbm.at[pl.ds(ebase0, W_CHUNKS * CHUNK)],
                        idx_v.at[pl.ds(0, W_CHUNKS * CHUNK)])

        def fill(g, cols_v):
            @plsc.parallel_loop(0, GROUP_ROWS // 16)
            def _blk(i16):
                e16 = (idx_v[pl.ds(g * GROUP_ROWS + i16 * 16, 16)] * EDGE_DIM
                       + lane_base)
                for j in range(EDGE_DIM):
                    cols_v[j, pl.ds(i16 * 16, 16)] = plsc.load_gather(
                        tbl_v, [e16 + cols[j]])

        def out_slice(g, n=GROUP_ROWS):
            return out_hbm.at[:, pl.ds(ebase0 + g * GROUP_ROWS, n)]

        def body(t, carry):
            @pl.when(t > 0)
            def _():
                pltpu.make_async_copy(cols_a, out_slice(0), sem_a).wait()
            fill(2 * t, cols_a)
            pltpu.async_copy(cols_a, out_slice(2 * t), sem_a)

            @pl.when(t > 0)
            def _():
                pltpu.make_async_copy(cols_b, out_slice(0), sem_b).wait()
            fill(2 * t + 1, cols_b)
            pltpu.async_copy(cols_b, out_slice(2 * t + 1), sem_b)
            return carry

        lax.fori_loop(0, GROUPS // 2, body, 0)
        pltpu.make_async_copy(cols_a, out_slice(0), sem_a).wait()
        pltpu.make_async_copy(cols_b, out_slice(0), sem_b).wait()

        @pl.when(wid < EXTRA)
        def _tail():
            pltpu.sync_copy(idx_hbm.at[pl.ds(ebase0 + W_CHUNKS * CHUNK,
                                             CHUNK)],
                            idx_v.at[pl.ds(W_CHUNKS * CHUNK, CHUNK)])

            @plsc.parallel_loop(0, CHUNK // 16)
            def _blk(i16):
                e16 = (idx_v[pl.ds(W_CHUNKS * CHUNK + i16 * 16, 16)]
                       * EDGE_DIM + lane_base)
                for j in range(EDGE_DIM):
                    cols_a[j, pl.ds(i16 * 16, 16)] = plsc.load_gather(
                        tbl_v, [e16 + cols[j]])

            pltpu.sync_copy(cols_a.at[:, pl.ds(0, CHUNK)],
                            out_hbm.at[:, pl.ds(ebase0 + W_CHUNKS * CHUNK,
                                                CHUNK)])

    return k(tbl_flat, edge_attr).T


def kernel(x_features, atoms, edge_attr, pos, atom_table, edge_table, W, b):
    del pos
    x_out = _node_proj(x_features, atoms.astype(jnp.int32), atom_table, W, b)
    edge_embed = _edge_gather(edge_table, edge_attr.astype(jnp.int32))
    return (x_out, edge_embed)


# speedup vs baseline: 1.1263x; 1.0361x over previous
import functools

import jax
import jax.numpy as jnp
from jax import lax
from jax.experimental import pallas as pl
from jax.experimental.pallas import tpu as pltpu
from jax.experimental.pallas import tpu_sc as plsc

NODE_DIM = 128
FT_DIM = 128
EDGE_DIM = 16
N_NODES = 100000
N_EDGES = 1600000
N_ATOM = 100

BN = 10000
NB = N_NODES // BN

CHUNK = 128
N_CHUNKS = N_EDGES // CHUNK
NW = 32
W_CHUNKS = N_CHUNKS // NW
EXTRA = N_CHUNKS - W_CHUNKS * NW
K = 13
GROUPS = W_CHUNKS // K
GROUP_ROWS = K * CHUNK


def _node_body(x_ref, atoms_ref, at_ref, w_ref, b_ref, o_ref):
    a = atoms_ref[0, 0, :]
    onehot = (a[:, None] == lax.broadcasted_iota(jnp.int32, (BN, N_ATOM), 1)
              ).astype(jnp.float32)
    w = w_ref[...]
    p = lax.dot_general(at_ref[...], w[:, :NODE_DIM],
                        (((1,), (1,)), ((), ())),
                        preferred_element_type=jnp.float32)
    t1 = lax.dot_general(onehot, p, (((1,), (0,)), ((), ())),
                         preferred_element_type=jnp.float32)
    t2 = lax.dot_general(x_ref[...], w[:, NODE_DIM:],
                         (((1,), (1,)), ((), ())),
                         preferred_element_type=jnp.float32)
    o_ref[...] = t1 + t2 + b_ref[...]


def _node_proj(x_features, atoms, atom_table, W, b):
    atoms3 = atoms.reshape(NB, 1, BN)
    b2 = b.reshape(1, NODE_DIM)
    return pl.pallas_call(
        _node_body,
        grid=(NB,),
        in_specs=[
            pl.BlockSpec((BN, FT_DIM), lambda i: (i, 0)),
            pl.BlockSpec((1, 1, BN), lambda i: (i, 0, 0)),
            pl.BlockSpec((N_ATOM, NODE_DIM), lambda i: (0, 0)),
            pl.BlockSpec((NODE_DIM, NODE_DIM + FT_DIM), lambda i: (0, 0)),
            pl.BlockSpec((1, NODE_DIM), lambda i: (0, 0)),
        ],
        out_specs=pl.BlockSpec((BN, NODE_DIM), lambda i: (i, 0)),
        out_shape=jax.ShapeDtypeStruct((N_NODES, NODE_DIM), jnp.float32),
    )(x_features, atoms3, atom_table, W, b2)


REP_STRIDE = 50 * EDGE_DIM + 1


def _edge_gather(edge_table, edge_attr):
    rep = jnp.broadcast_to(edge_table.reshape(1, 50 * EDGE_DIM),
                           (16, 50 * EDGE_DIM))
    tbl_flat = jnp.pad(rep, ((0, 0), (0, 1))).reshape(16 * REP_STRIDE)
    mesh = plsc.VectorSubcoreMesh(core_axis_name="c", subcore_axis_name="s")

    @functools.partial(
        pl.kernel,
        mesh=mesh,
        out_type=jax.ShapeDtypeStruct((EDGE_DIM, N_EDGES), jnp.float32),
        scratch_types=[
            pltpu.VMEM((16 * REP_STRIDE,), jnp.float32),
            pltpu.VMEM(((W_CHUNKS + 1) * CHUNK,), jnp.int32),
            pltpu.VMEM((EDGE_DIM, GROUP_ROWS), jnp.float32),
            pltpu.VMEM((EDGE_DIM, GROUP_ROWS), jnp.float32),
            pltpu.SemaphoreType.DMA,
            pltpu.SemaphoreType.DMA,
        ],
        compiler_params=pltpu.CompilerParams(needs_layout_passes=False),
    )
    def k(table_hbm, idx_hbm, out_hbm, tbl_v, idx_v, cols_a, cols_b,
          sem_a, sem_b):
        wid = lax.axis_index("s") * 2 + jax.lax.axis_index("c")
        cbase = wid * W_CHUNKS + jnp.minimum(wid, EXTRA)
        ebase0 = cbase * CHUNK
        lane_base = lax.iota(jnp.int32, 16) * REP_STRIDE
        cols = [jnp.full((16,), j, jnp.int32) for j in range(EDGE_DIM)]

        pltpu.sync_copy(table_hbm, tbl_v)
        pltpu.sync_copy(idx_hbm.at[pl.ds(ebase0, W_CHUNKS * CHUNK)],
                        idx_v.at[pl.ds(0, W_CHUNKS * CHUNK)])

        def fill(g, cols_v):
            @plsc.parallel_loop(0, GROUP_ROWS // 16)
            def _blk(i16):
                e16 = (idx_v[pl.ds(g * GROUP_ROWS + i16 * 16, 16)] * EDGE_DIM
                       + lane_base)
                for j in range(EDGE_DIM):
                    cols_v[j, pl.ds(i16 * 16, 16)] = plsc.load_gather(
                        tbl_v, [e16 + cols[j]])

        def out_slice(g, n=GROUP_ROWS):
            return out_hbm.at[:, pl.ds(ebase0 + g * GROUP_ROWS, n)]

        def body(t, carry):
            @pl.when(t > 0)
            def _():
                pltpu.make_async_copy(cols_a, out_slice(0), sem_a).wait()
            fill(2 * t, cols_a)
            pltpu.async_copy(cols_a, out_slice(2 * t), sem_a)

            @pl.when(t > 0)
            def _():
                pltpu.make_async_copy(cols_b, out_slice(0), sem_b).wait()
            fill(2 * t + 1, cols_b)
            pltpu.async_copy(cols_b, out_slice(2 * t + 1), sem_b)
            return carry

        lax.fori_loop(0, GROUPS // 2, body, 0)
        pltpu.make_async_copy(cols_a, out_slice(0), sem_a).wait()
        pltpu.make_async_copy(cols_b, out_slice(0), sem_b).wait()

        @pl.when(wid < EXTRA)
        def _tail():
            pltpu.sync_copy(idx_hbm.at[pl.ds(ebase0 + W_CHUNKS * CHUNK,
                                             CHUNK)],
                            idx_v.at[pl.ds(W_CHUNKS * CHUNK, CHUNK)])

            @plsc.parallel_loop(0, CHUNK // 16)
            def _blk(i16):
                e16 = (idx_v[pl.ds(W_CHUNKS * CHUNK + i16 * 16, 16)]
                       * EDGE_DIM + lane_base)
                for j in range(EDGE_DIM):
                    cols_a[j, pl.ds(i16 * 16, 16)] = plsc.load_gather(
                        tbl_v, [e16 + cols[j]])

            pltpu.sync_copy(cols_a.at[:, pl.ds(0, CHUNK)],
                            out_hbm.at[:, pl.ds(ebase0 + W_CHUNKS * CHUNK,
                                                CHUNK)])

    return k(tbl_flat, edge_attr).T


def kernel(x_features, atoms, edge_attr, pos, atom_table, edge_table, W, b):
    del pos
    x_out = _node_proj(x_features, atoms.astype(jnp.int32), atom_table, W, b)
    edge_embed = _edge_gather(edge_table, edge_attr.astype(jnp.int32))
    return (x_out, edge_embed)
